# Initial kernel scaffold; baseline (speedup 1.0000x reference)
import jax
import jax.numpy as jnp
from jax.experimental import pallas as pl


def kernel(emb, edge_index, W1, al1, ar1, b1, W2, al2, ar2, b2):
    raise NotImplementedError("write your pallas kernel here")


# speedup vs baseline: 31.8966x; 31.8966x over previous
import functools

import jax
import jax.numpy as jnp
import numpy as np
from jax import lax
from jax.experimental import pallas as pl
from jax.experimental.pallas import tpu as pltpu
from jax.experimental.pallas import tpu_sc as plsc

_N = 100000
_E = 1600000
_IN = 128
_HID = 8
_HEADS = 4
_OUT = 16

_RB = 1000
_EB = 128
_ZB = 200
_NZ = _N // _ZB
_NCH = -(-_N // _EB)
_TAIL = _N - (_NCH - 1) * _EB
_G = _EB // 16


def _block_diag_cols(a):
    h, f = a.shape
    eye = jnp.eye(h, dtype=a.dtype)
    return (a[:, :, None] * eye[:, None, :]).reshape(h * f, h)


def _featprep1_body(x_ref, w_ref, m_ref, feat_ref, elr_ref):
    f = lax.dot_general(x_ref[...], w_ref[...], (((1,), (1,)), ((), ())),
                        preferred_element_type=jnp.float32,
                        precision=lax.Precision.HIGHEST)
    feat_ref[...] = f
    elr_ref[...] = lax.dot_general(f, m_ref[...], (((1,), (0,)), ((), ())),
                                   preferred_element_type=jnp.float32,
                                   precision=lax.Precision.HIGHEST)


def _featprep1(emb, w1, melr1):
    c = _HEADS * _HID
    return pl.pallas_call(
        _featprep1_body,
        grid=(_N // _RB,),
        in_specs=[
            pl.BlockSpec((_RB, _IN), lambda i: (i, 0)),
            pl.BlockSpec((c, _IN), lambda i: (0, 0)),
            pl.BlockSpec((c, 16), lambda i: (0, 0)),
        ],
        out_specs=[
            pl.BlockSpec((_RB, c), lambda i: (i, 0)),
            pl.BlockSpec((_RB, 16), lambda i: (i, 0)),
        ],
        out_shape=[
            jax.ShapeDtypeStruct((_N, c), jnp.float32),
            jax.ShapeDtypeStruct((_N, 16), jnp.float32),
        ],
    )(emb, w1, melr1)


def _sc_edge_pass(src, dst, feat_tab, elr, layer):
    mesh = plsc.VectorSubcoreMesh(core_axis_name="c", subcore_axis_name="s")
    nblk = _E // _EB if layer == 1 else _E // _EB // 2

    @functools.partial(
        pl.kernel,
        out_type=(
            jax.ShapeDtypeStruct((2 * _N, 16), jnp.float32),
            jax.ShapeDtypeStruct((2 * _N, 16), jnp.float32),
        ),
        mesh=mesh,
        compiler_params=pltpu.CompilerParams(needs_layout_passes=False,
                                             use_tc_tiling_on_sc=False),
        scratch_types=(
            pltpu.VMEM_SHARED((_N, 16), jnp.float32),
            pltpu.VMEM((_EB,), jnp.int32),
            pltpu.VMEM((_EB,), jnp.int32),
            pltpu.VMEM((_EB,), jnp.int32),
            pltpu.VMEM((_EB, 16), jnp.float32),
            pltpu.VMEM((_EB, 16), jnp.float32),
            pltpu.VMEM((_EB, 16), jnp.float32),
            pltpu.VMEM((_EB, 16), jnp.float32),
            pltpu.VMEM((_ZB, 16), jnp.float32),
            pltpu.SemaphoreType.DMA,
        ),
    )
    def k(src_h, dst_h, feat_h, elr_h, num_h, den_h,
          acc_sh, src_v, dst_v, fidx_v, elrs_v, elrd_v, feat_v,
          msg_v, z16_v, sem):
        cid = lax.axis_index("c")
        sid = lax.axis_index("s")
        iota = lax.broadcasted_iota(jnp.int32, (16,), 0)
        zf = jnp.zeros((16,), jnp.float32)
        ci0 = jnp.zeros((16,), jnp.int32)

        def _z16(i, c):
            z16_v[i, :] = zf
            return c
        lax.fori_loop(0, _ZB, _z16, 0)

        def _zmsg(i, c):
            msg_v[i, :] = zf
            return c
        lax.fori_loop(0, _EB, _zmsg, 0)

        ha = cid * 2

        def _zero_acc():
            def _zsh(b, c):
                blk = sid + b * 16

                @pl.when(blk < _NZ)
                def _():
                    pltpu.sync_copy(z16_v, acc_sh.at[pl.ds(blk * _ZB, _ZB)])
                return c
            lax.fori_loop(0, -(-_NZ // 16), _zsh, 0)

        def _write_acc(out_h):
            def _wb(b, c):
                blk = sid + b * 16

                @pl.when(blk < _NZ)
                def _():
                    pltpu.sync_copy(acc_sh.at[pl.ds(blk * _ZB, _ZB)], z16_v)
                    pltpu.sync_copy(
                        z16_v, out_h.at[pl.ds(cid * _N + blk * _ZB, _ZB)])
                return c
            lax.fori_loop(0, -(-_NZ // 16), _wb, 0)
            lax.fori_loop(0, _ZB, _z16, 0)

        def _edge_base(blk):
            if layer == 1:
                return blk * _EB
            return (cid * nblk + blk) * _EB

        _zero_acc()
        plsc.subcore_barrier()

        def _blkden(b, c):
            blk = sid + b * 16

            @pl.when(blk < nblk)
            def _():
                base = _edge_base(blk)
                pltpu.sync_copy(src_h.at[pl.ds(base, _EB)], src_v)
                pltpu.sync_copy(dst_h.at[pl.ds(base, _EB)], dst_v)
                cp1 = pltpu.async_copy(elr_h.at[src_v], elrs_v, sem)
                cp2 = pltpu.async_copy(elr_h.at[dst_v], elrd_v, sem)
                cp1.wait()
                cp2.wait()

                if layer == 1:
                    def _grp(g, c2):
                        lane = g * 16 + iota
                        ca = ci0 + ha
                        els_a = plsc.load_gather(elrs_v, [lane, ca])
                        els_b = plsc.load_gather(elrs_v, [lane, ca + 1])
                        erd_a = plsc.load_gather(elrd_v, [lane, ca + 4])
                        erd_b = plsc.load_gather(elrd_v, [lane, ca + 5])
                        ea = els_a + erd_a
                        eb = els_b + erd_b
                        eea = jnp.exp(jnp.maximum(ea, ea * 0.2))
                        eeb = jnp.exp(jnp.maximum(eb, eb * 0.2))
                        plsc.store_scatter(msg_v, [lane, ci0], eea)
                        plsc.store_scatter(msg_v, [lane, ci0 + 1], eeb)
                        return c2
                else:
                    def _grp(g, c2):
                        lane = g * 16 + iota
                        el = plsc.load_gather(elrs_v, [lane, ci0])
                        er = plsc.load_gather(elrd_v, [lane, ci0 + 1])
                        e = el + er
                        ee = jnp.exp(jnp.maximum(e, e * 0.2))
                        plsc.store_scatter(msg_v, [lane, ci0], ee)
                        return c2
                lax.fori_loop(0, _G, _grp, 0)
                pltpu.sync_copy(msg_v, acc_sh.at[dst_v], add=True)
            return c
        lax.fori_loop(0, -(-nblk // 16), _blkden, 0)
        plsc.subcore_barrier()
        _write_acc(den_h)
        plsc.subcore_barrier()

        _zero_acc()
        plsc.subcore_barrier()

        def _blk(b, c):
            blk = sid + b * 16

            @pl.when(blk < nblk)
            def _():
                base = _edge_base(blk)
                pltpu.sync_copy(src_h.at[pl.ds(base, _EB)], src_v)
                pltpu.sync_copy(dst_h.at[pl.ds(base, _EB)], dst_v)

                if layer == 1:
                    def _fidx(g, c2):
                        lane = g * 16 + iota
                        v = plsc.load_gather(src_v, [lane])
                        plsc.store_scatter(fidx_v, [lane], v * 2 + cid)
                        return c2
                    lax.fori_loop(0, _G, _fidx, 0)
                    feat_idx = fidx_v
                else:
                    feat_idx = src_v

                cp1 = pltpu.async_copy(elr_h.at[src_v], elrs_v, sem)
                cp2 = pltpu.async_copy(elr_h.at[dst_v], elrd_v, sem)
                cp3 = pltpu.async_copy(feat_h.at[feat_idx], feat_v, sem)
                cp1.wait()
                cp2.wait()
                cp3.wait()

                if layer == 1:
                    def _grp(g, c2):
                        lane = g * 16 + iota
                        ca = ci0 + ha
                        els_a = plsc.load_gather(elrs_v, [lane, ca])
                        els_b = plsc.load_gather(elrs_v, [lane, ca + 1])
                        erd_a = plsc.load_gather(elrd_v, [lane, ca + 4])
                        erd_b = plsc.load_gather(elrd_v, [lane, ca + 5])
                        ea = els_a + erd_a
                        eb = els_b + erd_b
                        eea = jnp.exp(jnp.maximum(ea, ea * 0.2))
                        eeb = jnp.exp(jnp.maximum(eb, eb * 0.2))
                        for j in range(16):
                            f = plsc.load_gather(feat_v, [lane, ci0 + j])
                            plsc.store_scatter(msg_v, [lane, ci0 + j],
                                               f * (eea if j < 8 else eeb))
                        return c2
                else:
                    def _grp(g, c2):
                        lane = g * 16 + iota
                        el = plsc.load_gather(elrs_v, [lane, ci0])
                        er = plsc.load_gather(elrd_v, [lane, ci0 + 1])
                        e = el + er
                        ee = jnp.exp(jnp.maximum(e, e * 0.2))
                        for j in range(16):
                            f = plsc.load_gather(feat_v, [lane, ci0 + j])
                            plsc.store_scatter(msg_v, [lane, ci0 + j], f * ee)
                        return c2
                lax.fori_loop(0, _G, _grp, 0)
                pltpu.sync_copy(msg_v, acc_sh.at[dst_v], add=True)
            return c
        lax.fori_loop(0, -(-nblk // 16), _blk, 0)
        plsc.subcore_barrier()
        _write_acc(num_h)

    return k(src, dst, feat_tab, elr)


def _mid_body(n0_ref, n1_ref, d0_ref, d1_ref, b1_ref, w2a_ref, w2b_ref,
              s01_ref, melr2_ref, feat2_ref, elr2_ref):
    s01 = s01_ref[...]
    d0 = d0_ref[...][:, :2]
    d1 = d1_ref[...][:, :2]
    r0 = jnp.where(d0 > 0, 1.0 / d0, 0.0)
    r1 = jnp.where(d1 > 0, 1.0 / d1, 0.0)
    e0 = lax.dot_general(r0, s01, (((1,), (0,)), ((), ())),
                         preferred_element_type=jnp.float32,
                         precision=lax.Precision.HIGHEST)
    e1 = lax.dot_general(r1, s01, (((1,), (0,)), ((), ())),
                         preferred_element_type=jnp.float32,
                         precision=lax.Precision.HIGHEST)
    b1v = b1_ref[...]
    h0 = n0_ref[...] * e0 + b1v[:, :16]
    h1 = n1_ref[...] * e1 + b1v[:, 16:]
    h0 = jnp.where(h0 > 0, h0, jnp.exp(h0) - 1.0)
    h1 = jnp.where(h1 > 0, h1, jnp.exp(h1) - 1.0)
    f2 = (lax.dot_general(h0, w2a_ref[...], (((1,), (1,)), ((), ())),
                          preferred_element_type=jnp.float32,
                          precision=lax.Precision.HIGHEST)
          + lax.dot_general(h1, w2b_ref[...], (((1,), (1,)), ((), ())),
                            preferred_element_type=jnp.float32,
                            precision=lax.Precision.HIGHEST))
    feat2_ref[...] = f2
    elr2_ref[...] = lax.dot_general(f2, melr2_ref[...], (((1,), (0,)), ((), ())),
                                    preferred_element_type=jnp.float32,
                                    precision=lax.Precision.HIGHEST)


def _mid(num0, num1, den0, den1, b1, w2a, w2b, s01, melr2):
    return pl.pallas_call(
        _mid_body,
        grid=(_N // _RB,),
        in_specs=[
            pl.BlockSpec((_RB, 16), lambda i: (i, 0)),
            pl.BlockSpec((_RB, 16), lambda i: (i + _N // _RB, 0)),
            pl.BlockSpec((_RB, 16), lambda i: (i, 0)),
            pl.BlockSpec((_RB, 16), lambda i: (i + _N // _RB, 0)),
            pl.BlockSpec((1, 32), lambda i: (0, 0)),
            pl.BlockSpec((16, 16), lambda i: (0, 0)),
            pl.BlockSpec((16, 16), lambda i: (0, 0)),
            pl.BlockSpec((2, 16), lambda i: (0, 0)),
            pl.BlockSpec((16, 16), lambda i: (0, 0)),
        ],
        out_specs=[
            pl.BlockSpec((_RB, 16), lambda i: (i, 0)),
            pl.BlockSpec((_RB, 16), lambda i: (i, 0)),
        ],
        out_shape=[
            jax.ShapeDtypeStruct((_N, 16), jnp.float32),
            jax.ShapeDtypeStruct((_N, 16), jnp.float32),
        ],
    )(num0, num1, den0, den1, b1, w2a, w2b, s01, melr2)


def _final_body(n0_ref, n1_ref, d0_ref, d1_ref, b2_ref, out_ref):
    num = n0_ref[...] + n1_ref[...]
    dsum = d0_ref[...][:, 0:1] + d1_ref[...][:, 0:1]
    r = jnp.where(dsum > 0, 1.0 / dsum, 0.0)
    out_ref[...] = num * r + b2_ref[...]


def _final(num0, num1, den0, den1, b2):
    return pl.pallas_call(
        _final_body,
        grid=(_N // _RB,),
        in_specs=[
            pl.BlockSpec((_RB, 16), lambda i: (i, 0)),
            pl.BlockSpec((_RB, 16), lambda i: (i + _N // _RB, 0)),
            pl.BlockSpec((_RB, 16), lambda i: (i, 0)),
            pl.BlockSpec((_RB, 16), lambda i: (i + _N // _RB, 0)),
            pl.BlockSpec((1, 16), lambda i: (0, 0)),
        ],
        out_specs=pl.BlockSpec((_RB, 16), lambda i: (i, 0)),
        out_shape=jax.ShapeDtypeStruct((_N, 16), jnp.float32),
    )(num0, num1, den0, den1, b2)


def kernel(emb, edge_index, W1, al1, ar1, b1, W2, al2, ar2, b2):
    src = edge_index[0]
    dst = edge_index[1]

    melr1 = jnp.concatenate(
        [_block_diag_cols(al1.reshape(_HEADS, _HID)),
         _block_diag_cols(ar1.reshape(_HEADS, _HID)),
         jnp.zeros((_HEADS * _HID, 8), jnp.float32)], axis=1)
    melr2 = jnp.concatenate(
        [al2.reshape(_OUT, 1), ar2.reshape(_OUT, 1),
         jnp.zeros((_OUT, 14), jnp.float32)], axis=1)
    s01 = jnp.asarray(np.kron(np.eye(2, dtype=np.float32),
                              np.ones((1, 8), np.float32)))

    feat1, elr1 = _featprep1(emb, W1, melr1)
    num_a, den_a = _sc_edge_pass(
        src, dst, feat1.reshape(2 * _N, 16), elr1, layer=1)
    feat2, elr2 = _mid(num_a, num_a, den_a, den_a, b1.reshape(1, 32),
                       W2[:, :16], W2[:, 16:], s01, melr2)
    num_b, den_b = _sc_edge_pass(src, dst, feat2, elr2, layer=2)
    return _final(num_b, num_b, den_b, den_b, b2.reshape(1, 16))


# speedup vs baseline: 36.4396x; 1.1424x over previous
import functools

import jax
import jax.numpy as jnp
import numpy as np
from jax import lax
from jax.experimental import pallas as pl
from jax.experimental.pallas import tpu as pltpu
from jax.experimental.pallas import tpu_sc as plsc

_N = 100000
_E = 1600000
_IN = 128
_HID = 8
_HEADS = 4
_OUT = 16

_RB = 1000
_EB = 128
_ZB = 200
_NZ = _N // _ZB
_NCH = -(-_N // _EB)
_TAIL = _N - (_NCH - 1) * _EB
_G = _EB // 16


def _block_diag_cols(a):
    h, f = a.shape
    eye = jnp.eye(h, dtype=a.dtype)
    return (a[:, :, None] * eye[:, None, :]).reshape(h * f, h)


def _featprep1_body(x_ref, w_ref, m_ref, feat_ref, elr_ref):
    f = lax.dot_general(x_ref[...], w_ref[...], (((1,), (1,)), ((), ())),
                        preferred_element_type=jnp.float32,
                        precision=lax.Precision.HIGHEST)
    feat_ref[...] = f
    elr_ref[...] = lax.dot_general(f, m_ref[...], (((1,), (0,)), ((), ())),
                                   preferred_element_type=jnp.float32,
                                   precision=lax.Precision.HIGHEST)


def _featprep1(emb, w1, melr1):
    c = _HEADS * _HID
    return pl.pallas_call(
        _featprep1_body,
        grid=(_N // _RB,),
        in_specs=[
            pl.BlockSpec((_RB, _IN), lambda i: (i, 0)),
            pl.BlockSpec((c, _IN), lambda i: (0, 0)),
            pl.BlockSpec((c, 16), lambda i: (0, 0)),
        ],
        out_specs=[
            pl.BlockSpec((_RB, c), lambda i: (i, 0)),
            pl.BlockSpec((_RB, 16), lambda i: (i, 0)),
        ],
        out_shape=[
            jax.ShapeDtypeStruct((_N, c), jnp.float32),
            jax.ShapeDtypeStruct((_N, 16), jnp.float32),
        ],
    )(emb, w1, melr1)


def _sc_edge_pass(src, dst, feat_tab, elr, layer):
    mesh = plsc.VectorSubcoreMesh(core_axis_name="c", subcore_axis_name="s")
    nblk = _E // _EB if layer == 1 else _E // _EB // 2

    @functools.partial(
        pl.kernel,
        out_type=(
            jax.ShapeDtypeStruct((2 * _N, 16), jnp.float32),
            jax.ShapeDtypeStruct((2 * _N, 16), jnp.float32),
        ),
        mesh=mesh,
        compiler_params=pltpu.CompilerParams(needs_layout_passes=False,
                                             use_tc_tiling_on_sc=False),
        scratch_types=(
            pltpu.VMEM_SHARED((_N, 16), jnp.float32),
            pltpu.VMEM((_EB,), jnp.int32),
            pltpu.VMEM((_EB,), jnp.int32),
            pltpu.VMEM((_EB,), jnp.int32),
            pltpu.VMEM((_EB, 16), jnp.float32),
            pltpu.VMEM((_EB, 16), jnp.float32),
            pltpu.VMEM((_EB, 16), jnp.float32),
            pltpu.VMEM((_EB, 16), jnp.float32),
            pltpu.VMEM((_ZB, 16), jnp.float32),
            pltpu.SemaphoreType.DMA,
        ),
    )
    def k(src_h, dst_h, feat_h, elr_h, num_h, den_h,
          acc_sh, src_v, dst_v, fidx_v, elrs_v, elrd_v, feat_v,
          msg_v, z16_v, sem):
        cid = lax.axis_index("c")
        sid = lax.axis_index("s")
        iota = lax.broadcasted_iota(jnp.int32, (16,), 0)
        zf = jnp.zeros((16,), jnp.float32)
        ci0 = jnp.zeros((16,), jnp.int32)

        def _z16(i, c):
            z16_v[i, :] = zf
            return c
        lax.fori_loop(0, _ZB, _z16, 0)

        def _zmsg(i, c):
            msg_v[i, :] = zf
            return c
        lax.fori_loop(0, _EB, _zmsg, 0)

        ha = cid * 2

        def _zero_acc():
            def _zsh(b, c):
                blk = sid + b * 16

                @pl.when(blk < _NZ)
                def _():
                    pltpu.sync_copy(z16_v, acc_sh.at[pl.ds(blk * _ZB, _ZB)])
                return c
            lax.fori_loop(0, -(-_NZ // 16), _zsh, 0)

        def _write_acc(out_h):
            def _wb(b, c):
                blk = sid + b * 16

                @pl.when(blk < _NZ)
                def _():
                    pltpu.sync_copy(acc_sh.at[pl.ds(blk * _ZB, _ZB)], z16_v)
                    pltpu.sync_copy(
                        z16_v, out_h.at[pl.ds(cid * _N + blk * _ZB, _ZB)])
                return c
            lax.fori_loop(0, -(-_NZ // 16), _wb, 0)
            lax.fori_loop(0, _ZB, _z16, 0)

        def _edge_base(blk):
            if layer == 1:
                return blk * _EB
            return (cid * nblk + blk) * _EB

        _zero_acc()
        plsc.subcore_barrier()

        def _blkden(b, c):
            blk = sid + b * 16

            @pl.when(blk < nblk)
            def _():
                base = _edge_base(blk)
                cp0a = pltpu.async_copy(src_h.at[pl.ds(base, _EB)], src_v, sem)
                cp0b = pltpu.async_copy(dst_h.at[pl.ds(base, _EB)], dst_v, sem)
                cp0a.wait()
                cp0b.wait()
                cp1 = pltpu.async_copy(elr_h.at[src_v], elrs_v, sem)
                cp2 = pltpu.async_copy(elr_h.at[dst_v], elrd_v, sem)
                cp1.wait()
                cp2.wait()

                if layer == 1:
                    def _grp(g, c2):
                        lane = g * 16 + iota
                        ca = ci0 + ha
                        els_a = plsc.load_gather(elrs_v, [lane, ca])
                        els_b = plsc.load_gather(elrs_v, [lane, ca + 1])
                        erd_a = plsc.load_gather(elrd_v, [lane, ca + 4])
                        erd_b = plsc.load_gather(elrd_v, [lane, ca + 5])
                        ea = els_a + erd_a
                        eb = els_b + erd_b
                        eea = jnp.exp(jnp.maximum(ea, ea * 0.2))
                        eeb = jnp.exp(jnp.maximum(eb, eb * 0.2))
                        plsc.store_scatter(msg_v, [lane, ci0], eea)
                        plsc.store_scatter(msg_v, [lane, ci0 + 1], eeb)
                        return c2
                else:
                    def _grp(g, c2):
                        lane = g * 16 + iota
                        el = plsc.load_gather(elrs_v, [lane, ci0])
                        er = plsc.load_gather(elrd_v, [lane, ci0 + 1])
                        e = el + er
                        ee = jnp.exp(jnp.maximum(e, e * 0.2))
                        plsc.store_scatter(msg_v, [lane, ci0], ee)
                        return c2
                lax.fori_loop(0, _G, _grp, 0)
                pltpu.sync_copy(msg_v, acc_sh.at[dst_v], add=True)
            return c
        lax.fori_loop(0, -(-nblk // 16), _blkden, 0)
        plsc.subcore_barrier()
        _write_acc(den_h)
        plsc.subcore_barrier()

        _zero_acc()
        plsc.subcore_barrier()

        def _blk(b, c):
            blk = sid + b * 16

            @pl.when(blk < nblk)
            def _():
                base = _edge_base(blk)
                cp0a = pltpu.async_copy(src_h.at[pl.ds(base, _EB)], src_v, sem)
                cp0b = pltpu.async_copy(dst_h.at[pl.ds(base, _EB)], dst_v, sem)
                cp0a.wait()
                cp0b.wait()

                if layer == 1:
                    def _fidx(g, c2):
                        lane = g * 16 + iota
                        v = plsc.load_gather(src_v, [lane])
                        plsc.store_scatter(fidx_v, [lane], v * 2 + cid)
                        return c2
                    lax.fori_loop(0, _G, _fidx, 0)
                    feat_idx = fidx_v
                else:
                    feat_idx = src_v

                cp1 = pltpu.async_copy(elr_h.at[src_v], elrs_v, sem)
                cp2 = pltpu.async_copy(elr_h.at[dst_v], elrd_v, sem)
                cp3 = pltpu.async_copy(feat_h.at[feat_idx], feat_v, sem)
                cp1.wait()
                cp2.wait()
                cp3.wait()

                if layer == 1:
                    def _grp(g, c2):
                        lane = g * 16 + iota
                        ca = ci0 + ha
                        els_a = plsc.load_gather(elrs_v, [lane, ca])
                        els_b = plsc.load_gather(elrs_v, [lane, ca + 1])
                        erd_a = plsc.load_gather(elrd_v, [lane, ca + 4])
                        erd_b = plsc.load_gather(elrd_v, [lane, ca + 5])
                        ea = els_a + erd_a
                        eb = els_b + erd_b
                        eea = jnp.exp(jnp.maximum(ea, ea * 0.2))
                        eeb = jnp.exp(jnp.maximum(eb, eb * 0.2))
                        for j in range(16):
                            f = plsc.load_gather(feat_v, [lane, ci0 + j])
                            plsc.store_scatter(msg_v, [lane, ci0 + j],
                                               f * (eea if j < 8 else eeb))
                        return c2
                else:
                    def _grp(g, c2):
                        lane = g * 16 + iota
                        el = plsc.load_gather(elrs_v, [lane, ci0])
                        er = plsc.load_gather(elrd_v, [lane, ci0 + 1])
                        e = el + er
                        ee = jnp.exp(jnp.maximum(e, e * 0.2))
                        for j in range(16):
                            f = plsc.load_gather(feat_v, [lane, ci0 + j])
                            plsc.store_scatter(msg_v, [lane, ci0 + j], f * ee)
                        return c2
                lax.fori_loop(0, _G, _grp, 0)
                pltpu.sync_copy(msg_v, acc_sh.at[dst_v], add=True)
            return c
        lax.fori_loop(0, -(-nblk // 16), _blk, 0)
        plsc.subcore_barrier()
        _write_acc(num_h)

    return k(src, dst, feat_tab, elr)


def _mid_body(n0_ref, n1_ref, d0_ref, d1_ref, b1_ref, w2a_ref, w2b_ref,
              s01_ref, melr2_ref, feat2_ref, elr2_ref):
    s01 = s01_ref[...]
    d0 = d0_ref[...][:, :2]
    d1 = d1_ref[...][:, :2]
    r0 = jnp.where(d0 > 0, 1.0 / d0, 0.0)
    r1 = jnp.where(d1 > 0, 1.0 / d1, 0.0)
    e0 = lax.dot_general(r0, s01, (((1,), (0,)), ((), ())),
                         preferred_element_type=jnp.float32,
                         precision=lax.Precision.HIGHEST)
    e1 = lax.dot_general(r1, s01, (((1,), (0,)), ((), ())),
                         preferred_element_type=jnp.float32,
                         precision=lax.Precision.HIGHEST)
    b1v = b1_ref[...]
    h0 = n0_ref[...] * e0 + b1v[:, :16]
    h1 = n1_ref[...] * e1 + b1v[:, 16:]
    h0 = jnp.where(h0 > 0, h0, jnp.exp(h0) - 1.0)
    h1 = jnp.where(h1 > 0, h1, jnp.exp(h1) - 1.0)
    f2 = (lax.dot_general(h0, w2a_ref[...], (((1,), (1,)), ((), ())),
                          preferred_element_type=jnp.float32,
                          precision=lax.Precision.HIGHEST)
          + lax.dot_general(h1, w2b_ref[...], (((1,), (1,)), ((), ())),
                            preferred_element_type=jnp.float32,
                            precision=lax.Precision.HIGHEST))
    feat2_ref[...] = f2
    elr2_ref[...] = lax.dot_general(f2, melr2_ref[...], (((1,), (0,)), ((), ())),
                                    preferred_element_type=jnp.float32,
                                    precision=lax.Precision.HIGHEST)


def _mid(num0, num1, den0, den1, b1, w2a, w2b, s01, melr2):
    return pl.pallas_call(
        _mid_body,
        grid=(_N // _RB,),
        in_specs=[
            pl.BlockSpec((_RB, 16), lambda i: (i, 0)),
            pl.BlockSpec((_RB, 16), lambda i: (i + _N // _RB, 0)),
            pl.BlockSpec((_RB, 16), lambda i: (i, 0)),
            pl.BlockSpec((_RB, 16), lambda i: (i + _N // _RB, 0)),
            pl.BlockSpec((1, 32), lambda i: (0, 0)),
            pl.BlockSpec((16, 16), lambda i: (0, 0)),
            pl.BlockSpec((16, 16), lambda i: (0, 0)),
            pl.BlockSpec((2, 16), lambda i: (0, 0)),
            pl.BlockSpec((16, 16), lambda i: (0, 0)),
        ],
        out_specs=[
            pl.BlockSpec((_RB, 16), lambda i: (i, 0)),
            pl.BlockSpec((_RB, 16), lambda i: (i, 0)),
        ],
        out_shape=[
            jax.ShapeDtypeStruct((_N, 16), jnp.float32),
            jax.ShapeDtypeStruct((_N, 16), jnp.float32),
        ],
    )(num0, num1, den0, den1, b1, w2a, w2b, s01, melr2)


def _final_body(n0_ref, n1_ref, d0_ref, d1_ref, b2_ref, out_ref):
    num = n0_ref[...] + n1_ref[...]
    dsum = d0_ref[...][:, 0:1] + d1_ref[...][:, 0:1]
    r = jnp.where(dsum > 0, 1.0 / dsum, 0.0)
    out_ref[...] = num * r + b2_ref[...]


def _final(num0, num1, den0, den1, b2):
    return pl.pallas_call(
        _final_body,
        grid=(_N // _RB,),
        in_specs=[
            pl.BlockSpec((_RB, 16), lambda i: (i, 0)),
            pl.BlockSpec((_RB, 16), lambda i: (i + _N // _RB, 0)),
            pl.BlockSpec((_RB, 16), lambda i: (i, 0)),
            pl.BlockSpec((_RB, 16), lambda i: (i + _N // _RB, 0)),
            pl.BlockSpec((1, 16), lambda i: (0, 0)),
        ],
        out_specs=pl.BlockSpec((_RB, 16), lambda i: (i, 0)),
        out_shape=jax.ShapeDtypeStruct((_N, 16), jnp.float32),
    )(num0, num1, den0, den1, b2)


def kernel(emb, edge_index, W1, al1, ar1, b1, W2, al2, ar2, b2):
    src = edge_index[0]
    dst = edge_index[1]

    melr1 = jnp.concatenate(
        [_block_diag_cols(al1.reshape(_HEADS, _HID)),
         _block_diag_cols(ar1.reshape(_HEADS, _HID)),
         jnp.zeros((_HEADS * _HID, 8), jnp.float32)], axis=1)
    melr2 = jnp.concatenate(
        [al2.reshape(_OUT, 1), ar2.reshape(_OUT, 1),
         jnp.zeros((_OUT, 14), jnp.float32)], axis=1)
    s01 = jnp.asarray(np.kron(np.eye(2, dtype=np.float32),
                              np.ones((1, 8), np.float32)))

    feat1, elr1 = _featprep1(emb, W1, melr1)
    num_a, den_a = _sc_edge_pass(
        src, dst, feat1.reshape(2 * _N, 16), elr1, layer=1)
    feat2, elr2 = _mid(num_a, num_a, den_a, den_a, b1.reshape(1, 32),
                       W2[:, :16], W2[:, 16:], s01, melr2)
    num_b, den_b = _sc_edge_pass(src, dst, feat2, elr2, layer=2)
    return _final(num_b, num_b, den_b, den_b, b2.reshape(1, 16))
